# Initial kernel scaffold; baseline (speedup 1.0000x reference)
import jax
import jax.numpy as jnp
from jax.experimental import pallas as pl


def kernel(x, table):
    raise NotImplementedError("write your pallas kernel here")


# speedup vs baseline: 78.0792x; 78.0792x over previous
import jax
import jax.numpy as jnp
from jax.experimental import pallas as pl
from jax.experimental.pallas import tpu as pltpu


def _lut_body(nchan, nval, ndim, xt_ref, tab_ref, out_ref):
    for c in range(nchan):
        xc = xt_ref[c]
        masks = [xc == v for v in range(nval - 1)]
        for d in range(ndim):
            acc = jnp.full(xc.shape, tab_ref[nval * c + nval - 1, d],
                           dtype=jnp.float32)
            for v in range(nval - 2, -1, -1):
                acc = jnp.where(masks[v], tab_ref[nval * c + v, d], acc)
            out_ref[:, ndim * c + d, :] = acc


def kernel(x, table):
    B, H, W, C = x.shape
    NE, D = table.shape
    NV = NE // C
    P = H * W
    BB = 16

    xt = jnp.transpose(x.reshape(B, P, C), (2, 0, 1))

    import functools
    body = functools.partial(_lut_body, C, NV, D)

    out = pl.pallas_call(
        body,
        grid=(B // BB,),
        in_specs=[
            pl.BlockSpec((C, BB, P), lambda i: (0, i, 0)),
            pl.BlockSpec(memory_space=pltpu.SMEM),
        ],
        out_specs=pl.BlockSpec((BB, C * D, P), lambda i: (i, 0, 0)),
        out_shape=jax.ShapeDtypeStruct((B, C * D, P), jnp.float32),
    )(xt, table)

    return out.reshape(B, C * D, H, W)


# speedup vs baseline: 84.9256x; 1.0877x over previous
import functools

import jax
import jax.numpy as jnp
from jax.experimental import pallas as pl
from jax.experimental.pallas import tpu as pltpu


def _lut_body(nchan, nval, ndim, xp_ref, tab_ref, out_ref):
    xp = xp_ref[...]
    for c in range(nchan):
        xc = jax.lax.shift_right_logical(xp, 4 * c) & 15
        accs = [jnp.full(xc.shape, tab_ref[nval * c + nval - 1, d],
                         dtype=jnp.float32) for d in range(ndim)]
        for v in range(nval - 2, -1, -1):
            m = xc == v
            for d in range(ndim):
                accs[d] = jnp.where(m, tab_ref[nval * c + v, d], accs[d])
        for d in range(ndim):
            out_ref[:, ndim * c + d, :] = accs[d]


def kernel(x, table):
    B, H, W, C = x.shape
    NE, D = table.shape
    NV = NE // C
    P = H * W
    BB = 16

    shifts = jnp.array([1 << (4 * c) for c in range(C)], dtype=jnp.int32)
    xp = jnp.sum(x.reshape(B, P, C) * shifts, axis=-1, dtype=jnp.int32)

    body = functools.partial(_lut_body, C, NV, D)

    out = pl.pallas_call(
        body,
        grid=(B // BB,),
        in_specs=[
            pl.BlockSpec((BB, P), lambda i: (i, 0)),
            pl.BlockSpec(memory_space=pltpu.SMEM),
        ],
        out_specs=pl.BlockSpec((BB, C * D, P), lambda i: (i, 0, 0)),
        out_shape=jax.ShapeDtypeStruct((B, C * D, P), jnp.float32),
    )(xp, table)

    return out.reshape(B, C * D, H, W)
